# Initial kernel scaffold; baseline (speedup 1.0000x reference)
import jax
import jax.numpy as jnp
from jax.experimental import pallas as pl


def kernel(code_levels, W0, W1, W2, W3):
    raise NotImplementedError("write your pallas kernel here")


# speedup vs baseline: 1.4222x; 1.4222x over previous
import functools

import jax
import jax.numpy as jnp
from jax import lax
from jax.experimental import pallas as pl
from jax.experimental.pallas import tpu as pltpu
from jax.experimental.pallas import tpu_sc as plsc

N = 50000
NLEV = 4
DIMS = (16, 32, 64, 128)
OFFS = (0, 16, 48, 112)
DTOT = 240
NC, NS = 2, 16
NW = NC * NS
C = 160
NUM_FULL = N // C
TAIL = N - NUM_FULL * C
NUM_CHUNKS = NUM_FULL + (1 if TAIL else 0)
ITERS = -(-NUM_CHUNKS // NW)


def _body(clv, w0, w1, w2, w3, out, clv_v, i0, i1, i2, i3, r0, r1, r2, r3,
          sem):
    tables = (w0, w1, w2, w3)
    idx_v = (i0, i1, i2, i3)
    rows_v = (r0, r1, r2, r3)
    wid = lax.axis_index("s") * NC + lax.axis_index("c")

    def do_chunk(base, rows):
        pltpu.sync_copy(clv.at[pl.ds(base, rows), :],
                        clv_v.at[pl.ds(0, rows), :])
        for l in range(NLEV):
            col = jnp.full((16,), l, jnp.int32)
            for j in range(rows // 16):
                row_ids = lax.iota(jnp.int32, 16) + (j * 16)
                v = plsc.load_gather(clv_v, [row_ids, col])
                idx_v[l][pl.ds(j * 16, 16)] = v - 1
        copies = []
        for l in range(NLEV):
            src = tables[l].at[idx_v[l].at[pl.ds(0, rows)]]
            dst = rows_v[l].at[pl.ds(0, rows), :]
            copies.append(pltpu.async_copy(src, dst, sem))
        for cp in copies:
            cp.wait()
        for l in range(NLEV):
            pltpu.sync_copy(rows_v[l].at[pl.ds(0, rows), :],
                            out.at[pl.ds(base, rows),
                                   pl.ds(OFFS[l], DIMS[l])])

    def loop_body(t, carry):
        chunk = t * NW + wid
        pl.when(chunk < NUM_FULL)(lambda: do_chunk(chunk * C, C))
        if TAIL:
            pl.when(chunk == NUM_FULL)(lambda: do_chunk(NUM_FULL * C, TAIL))
        return carry

    lax.fori_loop(0, ITERS, loop_body, 0)


@jax.jit
def kernel(code_levels, W0, W1, W2, W3):
    mesh = plsc.VectorSubcoreMesh(core_axis_name="c", subcore_axis_name="s")
    f = pl.kernel(
        _body,
        out_type=jax.ShapeDtypeStruct((N, DTOT), jnp.float32),
        mesh=mesh,
        scratch_types=[
            pltpu.VMEM((C, NLEV), jnp.int32),
            pltpu.VMEM((C,), jnp.int32),
            pltpu.VMEM((C,), jnp.int32),
            pltpu.VMEM((C,), jnp.int32),
            pltpu.VMEM((C,), jnp.int32),
            pltpu.VMEM((C, DIMS[0]), jnp.float32),
            pltpu.VMEM((C, DIMS[1]), jnp.float32),
            pltpu.VMEM((C, DIMS[2]), jnp.float32),
            pltpu.VMEM((C, DIMS[3]), jnp.float32),
            pltpu.SemaphoreType.DMA,
        ],
        compiler_params=pltpu.CompilerParams(
            use_tc_tiling_on_sc=False, needs_layout_passes=False),
    )
    return f(code_levels, W0, W1, W2, W3)


# speedup vs baseline: 1.4350x; 1.0090x over previous
import functools

import jax
import jax.numpy as jnp
from jax import lax
from jax.experimental import pallas as pl
from jax.experimental.pallas import tpu as pltpu
from jax.experimental.pallas import tpu_sc as plsc

N = 50000
NLEV = 4
DIMS = (16, 32, 64, 128)
OFFS = (0, 16, 48, 112)
DTOT = 240
NC, NS = 2, 16
NW = NC * NS
C = 224
NUM_CHUNKS = -(-N // C)
ITERS = NUM_CHUNKS // NW


def _body(clv, w0, w1, w2, w3, out, clv_v, i0, i1, i2, i3, r0, r1, r2, r3,
          sem):
    tables = (w0, w1, w2, w3)
    idx_v = (i0, i1, i2, i3)
    rows_v = (r0, r1, r2, r3)
    wid = lax.axis_index("s") * NC + lax.axis_index("c")

    def do_chunk(base):
        pltpu.sync_copy(clv.at[pl.ds(base, C), :], clv_v)
        for l in range(NLEV):
            col = jnp.full((16,), l, jnp.int32)
            for j in range(C // 16):
                row_ids = lax.iota(jnp.int32, 16) + (j * 16)
                v = plsc.load_gather(clv_v, [row_ids, col])
                idx_v[l][pl.ds(j * 16, 16)] = v - 1
        copies = []
        for l in range(NLEV):
            copies.append(
                pltpu.async_copy(tables[l].at[idx_v[l]], rows_v[l], sem))
        for cp in copies:
            cp.wait()
        for l in range(NLEV):
            pltpu.sync_copy(rows_v[l],
                            out.at[pl.ds(base, C), pl.ds(OFFS[l], DIMS[l])])

    def loop_body(t, carry):
        chunk = t * NW + wid
        do_chunk(lax.min(chunk * C, N - C))
        return carry

    lax.fori_loop(0, ITERS, loop_body, 0)


@jax.jit
def kernel(code_levels, W0, W1, W2, W3):
    mesh = plsc.VectorSubcoreMesh(core_axis_name="c", subcore_axis_name="s")
    f = pl.kernel(
        _body,
        out_type=jax.ShapeDtypeStruct((N, DTOT), jnp.float32),
        mesh=mesh,
        scratch_types=[
            pltpu.VMEM((C, NLEV), jnp.int32),
            pltpu.VMEM((C,), jnp.int32),
            pltpu.VMEM((C,), jnp.int32),
            pltpu.VMEM((C,), jnp.int32),
            pltpu.VMEM((C,), jnp.int32),
            pltpu.VMEM((C, DIMS[0]), jnp.float32),
            pltpu.VMEM((C, DIMS[1]), jnp.float32),
            pltpu.VMEM((C, DIMS[2]), jnp.float32),
            pltpu.VMEM((C, DIMS[3]), jnp.float32),
            pltpu.SemaphoreType.DMA,
        ],
        compiler_params=pltpu.CompilerParams(
            use_tc_tiling_on_sc=False, needs_layout_passes=False),
    )
    return f(code_levels, W0, W1, W2, W3)


# speedup vs baseline: 1.5791x; 1.1004x over previous
import functools

import jax
import jax.numpy as jnp
from jax import lax
from jax.experimental import pallas as pl
from jax.experimental.pallas import tpu as pltpu
from jax.experimental.pallas import tpu_sc as plsc

N = 50000
NLEV = 4
DIMS = (16, 32, 64, 128)
OFFS = (0, 16, 48, 112)
DTOT = 240
NC, NS = 2, 16
NW = NC * NS
C = 224
NUM_CHUNKS = -(-N // C)
ITERS = NUM_CHUNKS // NW


def _body(clvt, w0, w1, w2, w3, out, i0, i1, i2, i3, r0, r1, r2, r3,
          sem):
    tables = (w0, w1, w2, w3)
    idx_v = (i0, i1, i2, i3)
    rows_v = (r0, r1, r2, r3)
    wid = lax.axis_index("s") * NC + lax.axis_index("c")

    def do_chunk(base):
        for l in range(NLEV):
            pltpu.sync_copy(clvt.at[l, pl.ds(base, C)], idx_v[l])
        for l in range(NLEV):
            for j in range(C // 16):
                v = idx_v[l][pl.ds(j * 16, 16)]
                idx_v[l][pl.ds(j * 16, 16)] = v - 1
        copies = []
        for l in range(NLEV):
            copies.append(
                pltpu.async_copy(tables[l].at[idx_v[l]], rows_v[l], sem))
        for cp in copies:
            cp.wait()
        for l in range(NLEV):
            pltpu.sync_copy(rows_v[l],
                            out.at[pl.ds(base, C), pl.ds(OFFS[l], DIMS[l])])

    def loop_body(t, carry):
        chunk = t * NW + wid
        do_chunk(lax.min(chunk * C, N - C))
        return carry

    lax.fori_loop(0, ITERS, loop_body, 0)


@jax.jit
def kernel(code_levels, W0, W1, W2, W3):
    mesh = plsc.VectorSubcoreMesh(core_axis_name="c", subcore_axis_name="s")
    f = pl.kernel(
        _body,
        out_type=jax.ShapeDtypeStruct((N, DTOT), jnp.float32),
        mesh=mesh,
        scratch_types=[
            pltpu.VMEM((C,), jnp.int32),
            pltpu.VMEM((C,), jnp.int32),
            pltpu.VMEM((C,), jnp.int32),
            pltpu.VMEM((C,), jnp.int32),
            pltpu.VMEM((C, DIMS[0]), jnp.float32),
            pltpu.VMEM((C, DIMS[1]), jnp.float32),
            pltpu.VMEM((C, DIMS[2]), jnp.float32),
            pltpu.VMEM((C, DIMS[3]), jnp.float32),
            pltpu.SemaphoreType.DMA,
        ],
        compiler_params=pltpu.CompilerParams(
            use_tc_tiling_on_sc=False, needs_layout_passes=False),
    )
    return f(code_levels.T, W0, W1, W2, W3)
